# baseline (speedup 1.0000x reference)
import dataclasses
import functools

import jax
import jax.numpy as jnp
from jax import lax
from jax.experimental import pallas as pl
from jax.experimental.pallas import tpu as pltpu
from jax.experimental.pallas import tpu_sc as plsc

_N_TILES = 32
_LANES = 16


@functools.lru_cache(maxsize=None)
def _build_sc_r0(n_edges: int, n_nodes: int, rad_len: int):
    edges_per_tile = n_edges // _N_TILES
    chunk = 4000
    n_chunks = edges_per_tile // chunk
    assert edges_per_tile % chunk == 0 and chunk % _LANES == 0
    assert n_chunks % 2 == 0 and n_chunks >= 4

    def body(ei_hbm, z_hbm, rad_hbm, out_hbm,
             zrn_v, rad_v, s0, s1, r0_, r1_, o0, o1,
             zsem, ss0, ss1, sr0, sr1, so0, so1):
        wid = lax.axis_index("s") * 2 + lax.axis_index("c")
        base = wid * edges_per_tile
        s_bufs, r_bufs, o_bufs = (s0, s1), (r0_, r1_), (o0, o1)
        sem_s, sem_r, sem_o = (ss0, ss1), (sr0, sr1), (so0, so1)

        def start_in(jj, b):
            eb = base + jj * chunk
            pltpu.async_copy(ei_hbm.at[0, pl.ds(eb, chunk)], s_bufs[b], sem_s[b])
            pltpu.async_copy(ei_hbm.at[1, pl.ds(eb, chunk)], r_bufs[b], sem_r[b])

        def wait_in(b):
            pltpu.make_async_copy(
                ei_hbm.at[0, pl.ds(0, chunk)], s_bufs[b], sem_s[b]
            ).wait()
            pltpu.make_async_copy(
                ei_hbm.at[0, pl.ds(0, chunk)], r_bufs[b], sem_r[b]
            ).wait()

        def start_out(jj, b):
            eb = base + jj * chunk
            pltpu.async_copy(o_bufs[b], out_hbm.at[pl.ds(eb, chunk)], sem_o[b])

        def wait_out(b):
            pltpu.make_async_copy(
                o_bufs[b], out_hbm.at[pl.ds(0, chunk)], sem_o[b]
            ).wait()

        pltpu.sync_copy(rad_hbm, rad_v)
        pltpu.sync_copy(z_hbm, zrn_v)
        start_in(0, 0)
        start_in(1, 1)

        @pl.loop(0, n_chunks, step=2)
        def _chunks(j):
            for b in range(2):
                jj = j + b

                @pl.when(jj >= 2)
                def _():
                    wait_out(b)

                wait_in(b)

                @plsc.parallel_loop(0, chunk, _LANES, unroll=8)
                def _vec(i):
                    sv = s_bufs[b][pl.ds(i, _LANES)]
                    rv = r_bufs[b][pl.ds(i, _LANES)]
                    zs = plsc.load_gather(zrn_v, [sv])
                    zr = plsc.load_gather(zrn_v, [rv])
                    rs = plsc.load_gather(rad_v, [zs])
                    rr = plsc.load_gather(rad_v, [zr])
                    o_bufs[b][pl.ds(i, _LANES)] = rs + rr

                start_out(jj, b)
                start_in(jnp.minimum(jj + 2, n_chunks - 1), b)

        wait_in(0)
        wait_in(1)
        wait_out(0)
        wait_out(1)

    cp = pltpu.CompilerParams(
        needs_layout_passes=False, use_tc_tiling_on_sc=False
    )
    return pl.kernel(
        body,
        out_type=jax.ShapeDtypeStruct((n_edges,), jnp.float32),
        compiler_params=cp,
        mesh=plsc.VectorSubcoreMesh(
            core_axis_name="c", subcore_axis_name="s",
            num_cores=2, num_subcores=16,
        ),
        scratch_types=[
            pltpu.VMEM((n_nodes,), jnp.int32),
            pltpu.VMEM((rad_len,), jnp.float32),
            pltpu.VMEM((chunk,), jnp.int32),
            pltpu.VMEM((chunk,), jnp.int32),
            pltpu.VMEM((chunk,), jnp.int32),
            pltpu.VMEM((chunk,), jnp.int32),
            pltpu.VMEM((chunk,), jnp.float32),
            pltpu.VMEM((chunk,), jnp.float32),
            pltpu.SemaphoreType.DMA,
            pltpu.SemaphoreType.DMA,
            pltpu.SemaphoreType.DMA,
            pltpu.SemaphoreType.DMA,
            pltpu.SemaphoreType.DMA,
            pltpu.SemaphoreType.DMA,
            pltpu.SemaphoreType.DMA,
        ],
    )


def _tc_w_body(s_ref, d_ref, r0_ref, o_ref):
    t = d_ref[...] / r0_ref[...]
    lt = jnp.log(t)
    tq = jnp.exp(s_ref[1] * lt)
    tqp = jnp.exp(s_ref[2] * lt)
    atq = s_ref[0] * tq
    o_ref[...] = atq / (1.0 + tqp + atq)


@functools.lru_cache(maxsize=None)
def _build_tc_w(n_edges: int):
    cols = 128
    rows = n_edges // cols
    block_rows = 2000
    assert rows % block_rows == 0
    grid = rows // block_rows
    return pl.pallas_call(
        _tc_w_body,
        out_shape=jax.ShapeDtypeStruct((rows, cols), jnp.float32),
        grid=(grid,),
        in_specs=[
            pl.BlockSpec(memory_space=pltpu.SMEM),
            pl.BlockSpec((block_rows, cols), lambda i: (i, 0)),
            pl.BlockSpec((block_rows, cols), lambda i: (i, 0)),
        ],
        out_specs=pl.BlockSpec((block_rows, cols), lambda i: (i, 0)),
    )


def kernel(z, edge_distance, edge_index, q, p, covalent_radii):
    n_edges = edge_distance.shape[0]
    n_nodes = z.shape[0]
    pp = 2.0 * jax.nn.softplus(0.5 * p) + 1.0
    qq = 2.0 * jax.nn.softplus(0.5 * q) + 1.0
    a = -2.0 * (pp + qq - 2.0 * qq * pp) / (pp**2 + pp + qq**2 + qq)
    scalars = jnp.stack([a, qq, qq - pp]).astype(jnp.float32)

    rad_len = 128
    rad = jnp.concatenate(
        [covalent_radii.astype(jnp.float32),
         jnp.zeros((rad_len - covalent_radii.shape[0],), jnp.float32)]
    )
    r0 = _build_sc_r0(n_edges, n_nodes, rad_len)(edge_index, z, rad)

    cols = 128
    d2 = edge_distance.reshape(n_edges // cols, cols)
    r02 = r0.reshape(n_edges // cols, cols)
    w = _build_tc_w(n_edges)(scalars, d2, r02)
    return w.reshape(n_edges)


# speedup vs baseline: 1.0279x; 1.0279x over previous
import dataclasses
import functools

import jax
import jax.numpy as jnp
from jax import lax
from jax.experimental import pallas as pl
from jax.experimental.pallas import tpu as pltpu
from jax.experimental.pallas import tpu_sc as plsc

_N_TILES = 32
_LANES = 16


@functools.lru_cache(maxsize=None)
def _build_sc_r0(n_edges: int, n_nodes: int, rad_len: int):
    edges_per_tile = n_edges // _N_TILES
    chunk = 4000
    n_chunks = edges_per_tile // chunk
    assert edges_per_tile % chunk == 0 and chunk % _LANES == 0
    assert n_chunks % 2 == 0 and n_chunks >= 4

    def body(ei_hbm, z_hbm, rad_hbm, out_hbm,
             zrn_v, rad_v, s0, s1, r0_, r1_, o0, o1,
             zsem, ss0, ss1, sr0, sr1, so0, so1):
        wid = lax.axis_index("s") * 2 + lax.axis_index("c")
        base = wid * edges_per_tile
        s_bufs, r_bufs, o_bufs = (s0, s1), (r0_, r1_), (o0, o1)
        sem_s, sem_r, sem_o = (ss0, ss1), (sr0, sr1), (so0, so1)

        def start_in(jj, b):
            eb = base + jj * chunk
            pltpu.async_copy(ei_hbm.at[0, pl.ds(eb, chunk)], s_bufs[b], sem_s[b])
            pltpu.async_copy(ei_hbm.at[1, pl.ds(eb, chunk)], r_bufs[b], sem_r[b])

        def wait_in(b):
            pltpu.make_async_copy(
                ei_hbm.at[0, pl.ds(0, chunk)], s_bufs[b], sem_s[b]
            ).wait()
            pltpu.make_async_copy(
                ei_hbm.at[0, pl.ds(0, chunk)], r_bufs[b], sem_r[b]
            ).wait()

        def start_out(jj, b):
            eb = base + jj * chunk
            pltpu.async_copy(o_bufs[b], out_hbm.at[pl.ds(eb, chunk)], sem_o[b])

        def wait_out(b):
            pltpu.make_async_copy(
                o_bufs[b], out_hbm.at[pl.ds(0, chunk)], sem_o[b]
            ).wait()

        pltpu.sync_copy(rad_hbm, rad_v)
        pltpu.async_copy(z_hbm, zrn_v, zsem)
        start_in(0, 0)
        start_in(1, 1)
        pltpu.make_async_copy(z_hbm, zrn_v, zsem).wait()

        @plsc.parallel_loop(0, n_nodes, _LANES, unroll=8)
        def _rn(i):
            zv = zrn_v[pl.ds(i, _LANES)]
            rv = plsc.load_gather(rad_v, [zv])
            zrn_v[pl.ds(i, _LANES)] = plsc.bitcast(rv, jnp.int32)

        plsc.subcore_barrier()

        @pl.loop(0, n_chunks, step=2)
        def _chunks(j):
            for b in range(2):
                jj = j + b

                @pl.when(jj >= 2)
                def _():
                    wait_out(b)

                wait_in(b)

                @plsc.parallel_loop(0, chunk, _LANES, unroll=8)
                def _vec(i):
                    sv = s_bufs[b][pl.ds(i, _LANES)]
                    rv = r_bufs[b][pl.ds(i, _LANES)]
                    rs = plsc.bitcast(plsc.load_gather(zrn_v, [sv]), jnp.float32)
                    rr = plsc.bitcast(plsc.load_gather(zrn_v, [rv]), jnp.float32)
                    o_bufs[b][pl.ds(i, _LANES)] = rs + rr

                start_out(jj, b)
                start_in(jnp.minimum(jj + 2, n_chunks - 1), b)

        wait_in(0)
        wait_in(1)
        wait_out(0)
        wait_out(1)

    cp = pltpu.CompilerParams(
        needs_layout_passes=False, use_tc_tiling_on_sc=False
    )
    return pl.kernel(
        body,
        out_type=jax.ShapeDtypeStruct((n_edges,), jnp.float32),
        compiler_params=cp,
        mesh=plsc.VectorSubcoreMesh(
            core_axis_name="c", subcore_axis_name="s",
            num_cores=2, num_subcores=16,
        ),
        scratch_types=[
            pltpu.VMEM((n_nodes,), jnp.int32),
            pltpu.VMEM((rad_len,), jnp.float32),
            pltpu.VMEM((chunk,), jnp.int32),
            pltpu.VMEM((chunk,), jnp.int32),
            pltpu.VMEM((chunk,), jnp.int32),
            pltpu.VMEM((chunk,), jnp.int32),
            pltpu.VMEM((chunk,), jnp.float32),
            pltpu.VMEM((chunk,), jnp.float32),
            pltpu.SemaphoreType.DMA,
            pltpu.SemaphoreType.DMA,
            pltpu.SemaphoreType.DMA,
            pltpu.SemaphoreType.DMA,
            pltpu.SemaphoreType.DMA,
            pltpu.SemaphoreType.DMA,
            pltpu.SemaphoreType.DMA,
        ],
    )


def _tc_w_body(s_ref, d_ref, r0_ref, o_ref):
    t = d_ref[...] / r0_ref[...]
    lt = jnp.log(t)
    tq = jnp.exp(s_ref[1] * lt)
    tqp = jnp.exp(s_ref[2] * lt)
    atq = s_ref[0] * tq
    o_ref[...] = atq / (1.0 + tqp + atq)


@functools.lru_cache(maxsize=None)
def _build_tc_w(n_edges: int):
    cols = 128
    rows = n_edges // cols
    block_rows = 2000
    assert rows % block_rows == 0
    grid = rows // block_rows
    return pl.pallas_call(
        _tc_w_body,
        out_shape=jax.ShapeDtypeStruct((rows, cols), jnp.float32),
        grid=(grid,),
        in_specs=[
            pl.BlockSpec(memory_space=pltpu.SMEM),
            pl.BlockSpec((block_rows, cols), lambda i: (i, 0)),
            pl.BlockSpec((block_rows, cols), lambda i: (i, 0)),
        ],
        out_specs=pl.BlockSpec((block_rows, cols), lambda i: (i, 0)),
    )


def kernel(z, edge_distance, edge_index, q, p, covalent_radii):
    n_edges = edge_distance.shape[0]
    n_nodes = z.shape[0]
    pp = 2.0 * jax.nn.softplus(0.5 * p) + 1.0
    qq = 2.0 * jax.nn.softplus(0.5 * q) + 1.0
    a = -2.0 * (pp + qq - 2.0 * qq * pp) / (pp**2 + pp + qq**2 + qq)
    scalars = jnp.stack([a, qq, qq - pp]).astype(jnp.float32)

    rad_len = 128
    rad = jnp.concatenate(
        [covalent_radii.astype(jnp.float32),
         jnp.zeros((rad_len - covalent_radii.shape[0],), jnp.float32)]
    )
    r0 = _build_sc_r0(n_edges, n_nodes, rad_len)(edge_index, z, rad)

    cols = 128
    d2 = edge_distance.reshape(n_edges // cols, cols)
    r02 = r0.reshape(n_edges // cols, cols)
    w = _build_tc_w(n_edges)(scalars, d2, r02)
    return w.reshape(n_edges)


# speedup vs baseline: 1.2361x; 1.2026x over previous
import dataclasses
import functools

import jax
import jax.numpy as jnp
from jax import lax
from jax.experimental import pallas as pl
from jax.experimental.pallas import tpu as pltpu
from jax.experimental.pallas import tpu_sc as plsc

_N_TILES = 32
_LANES = 16


@functools.lru_cache(maxsize=None)
def _build_sc_r0(n_edges: int, n_nodes: int, rad_len: int):
    chunk = 2560
    n_chunks = n_edges // chunk
    assert n_edges % chunk == 0 and chunk % 128 == 0
    n_slots = n_chunks // _N_TILES + (1 if n_chunks % _N_TILES else 0)
    if n_slots % 2:
        n_slots += 1

    def body(ei_hbm, z_hbm, rad_hbm, out_hbm,
             zrn_v, rad_v, e0, e1, o0, o1,
             zsem, se0, se1, so0, so1):
        wid = lax.axis_index("s") * 2 + lax.axis_index("c")
        e_bufs, o_bufs = (e0, e1), (o0, o1)
        sem_e, sem_o = (se0, se1), (so0, so1)

        def start_in(cid, b):
            pltpu.async_copy(
                ei_hbm.at[:, pl.ds(cid * chunk, chunk)], e_bufs[b], sem_e[b]
            )

        def wait_in(b):
            pltpu.make_async_copy(
                ei_hbm.at[:, pl.ds(0, chunk)], e_bufs[b], sem_e[b]
            ).wait()

        def start_out(cid, b):
            pltpu.async_copy(
                o_bufs[b], out_hbm.at[pl.ds(cid * chunk, chunk)], sem_o[b]
            )

        def wait_out(b):
            pltpu.make_async_copy(
                o_bufs[b], out_hbm.at[pl.ds(0, chunk)], sem_o[b]
            ).wait()

        pltpu.sync_copy(rad_hbm, rad_v)
        pltpu.async_copy(z_hbm, zrn_v, zsem)
        start_in(wid, 0)
        start_in(wid + _N_TILES, 1)
        pltpu.make_async_copy(z_hbm, zrn_v, zsem).wait()

        @plsc.parallel_loop(0, n_nodes, _LANES, unroll=8)
        def _rn(i):
            zv = zrn_v[pl.ds(i, _LANES)]
            rv = plsc.load_gather(rad_v, [zv])
            zrn_v[pl.ds(i, _LANES)] = plsc.bitcast(rv, jnp.int32)

        plsc.subcore_barrier()

        @pl.loop(0, n_slots, step=2)
        def _slots(t):
            for b in range(2):
                cid = (t + b) * _N_TILES + wid

                @pl.when(cid < n_chunks)
                def _():
                    @pl.when(t + b >= 2)
                    def _():
                        wait_out(b)

                    wait_in(b)

                    @plsc.parallel_loop(0, chunk, _LANES, unroll=8)
                    def _vec(i):
                        sv = e_bufs[b][0, pl.ds(i, _LANES)]
                        rv = e_bufs[b][1, pl.ds(i, _LANES)]
                        rs = plsc.bitcast(
                            plsc.load_gather(zrn_v, [sv]), jnp.float32
                        )
                        rr = plsc.bitcast(
                            plsc.load_gather(zrn_v, [rv]), jnp.float32
                        )
                        o_bufs[b][pl.ds(i, _LANES)] = rs + rr

                    start_out(cid, b)
                    nxt = cid + 2 * _N_TILES

                    @pl.when(nxt < n_chunks)
                    def _():
                        start_in(nxt, b)

        wait_out(0)
        wait_out(1)

    cp = pltpu.CompilerParams(needs_layout_passes=False)
    return pl.kernel(
        body,
        out_type=jax.ShapeDtypeStruct((n_edges,), jnp.float32),
        compiler_params=cp,
        mesh=plsc.VectorSubcoreMesh(
            core_axis_name="c", subcore_axis_name="s",
            num_cores=2, num_subcores=16,
        ),
        scratch_types=[
            pltpu.VMEM((n_nodes,), jnp.int32),
            pltpu.VMEM((rad_len,), jnp.float32),
            pltpu.VMEM((2, chunk), jnp.int32),
            pltpu.VMEM((2, chunk), jnp.int32),
            pltpu.VMEM((chunk,), jnp.float32),
            pltpu.VMEM((chunk,), jnp.float32),
            pltpu.SemaphoreType.DMA,
            pltpu.SemaphoreType.DMA,
            pltpu.SemaphoreType.DMA,
            pltpu.SemaphoreType.DMA,
            pltpu.SemaphoreType.DMA,
        ],
    )


def _tc_w_body(s_ref, d_ref, r0_ref, o_ref):
    t = d_ref[...] / r0_ref[...]
    lt = jnp.log(t)
    tq = jnp.exp(s_ref[1] * lt)
    tqp = jnp.exp(s_ref[2] * lt)
    atq = s_ref[0] * tq
    o_ref[...] = atq / (1.0 + tqp + atq)


@functools.lru_cache(maxsize=None)
def _build_tc_w(n_edges: int):
    cols = 128
    rows = n_edges // cols
    block_rows = 2000
    assert rows % block_rows == 0
    grid = rows // block_rows
    return pl.pallas_call(
        _tc_w_body,
        out_shape=jax.ShapeDtypeStruct((rows, cols), jnp.float32),
        grid=(grid,),
        in_specs=[
            pl.BlockSpec(memory_space=pltpu.SMEM),
            pl.BlockSpec((block_rows, cols), lambda i: (i, 0)),
            pl.BlockSpec((block_rows, cols), lambda i: (i, 0)),
        ],
        out_specs=pl.BlockSpec((block_rows, cols), lambda i: (i, 0)),
    )


def kernel(z, edge_distance, edge_index, q, p, covalent_radii):
    n_edges = edge_distance.shape[0]
    n_nodes = z.shape[0]
    pp = 2.0 * jax.nn.softplus(0.5 * p) + 1.0
    qq = 2.0 * jax.nn.softplus(0.5 * q) + 1.0
    a = -2.0 * (pp + qq - 2.0 * qq * pp) / (pp**2 + pp + qq**2 + qq)
    scalars = jnp.stack([a, qq, qq - pp]).astype(jnp.float32)

    rad_len = 128
    rad = jnp.concatenate(
        [covalent_radii.astype(jnp.float32),
         jnp.zeros((rad_len - covalent_radii.shape[0],), jnp.float32)]
    )
    r0 = _build_sc_r0(n_edges, n_nodes, rad_len)(edge_index, z, rad)

    cols = 128
    d2 = edge_distance.reshape(n_edges // cols, cols)
    r02 = r0.reshape(n_edges // cols, cols)
    w = _build_tc_w(n_edges)(scalars, d2, r02)
    return w.reshape(n_edges)


# speedup vs baseline: 1.3694x; 1.1078x over previous
import dataclasses
import functools

import jax
import jax.numpy as jnp
from jax import lax
from jax.experimental import pallas as pl
from jax.experimental.pallas import tpu as pltpu
from jax.experimental.pallas import tpu_sc as plsc

_N_TILES = 32
_LANES = 16


@functools.lru_cache(maxsize=None)
def _build_sc_r0(n_edges: int, n_nodes: int, rad_len: int):
    chunk = 5120
    n_chunks = n_edges // chunk
    assert n_edges % chunk == 0 and chunk % 128 == 0
    n_slots = n_chunks // _N_TILES + (1 if n_chunks % _N_TILES else 0)
    if n_slots % 2:
        n_slots += 1

    def body(ei_hbm, z_hbm, rad_hbm, out_hbm,
             zrn_v, rad_v, e0, e1, o0, o1,
             zsem, se0, se1, so0, so1):
        wid = lax.axis_index("s") * 2 + lax.axis_index("c")
        e_bufs, o_bufs = (e0, e1), (o0, o1)
        sem_e, sem_o = (se0, se1), (so0, so1)

        def start_in(cid, b):
            pltpu.async_copy(
                ei_hbm.at[:, pl.ds(cid * chunk, chunk)], e_bufs[b], sem_e[b]
            )

        def wait_in(b):
            pltpu.make_async_copy(
                ei_hbm.at[:, pl.ds(0, chunk)], e_bufs[b], sem_e[b]
            ).wait()

        def start_out(cid, b):
            pltpu.async_copy(
                o_bufs[b], out_hbm.at[pl.ds(cid * chunk, chunk)], sem_o[b]
            )

        def wait_out(b):
            pltpu.make_async_copy(
                o_bufs[b], out_hbm.at[pl.ds(0, chunk)], sem_o[b]
            ).wait()

        pltpu.sync_copy(rad_hbm, rad_v)
        pltpu.async_copy(z_hbm, zrn_v, zsem)
        start_in(wid, 0)
        start_in(wid + _N_TILES, 1)
        pltpu.make_async_copy(z_hbm, zrn_v, zsem).wait()

        @plsc.parallel_loop(0, n_nodes, _LANES, unroll=8)
        def _rn(i):
            zv = zrn_v[pl.ds(i, _LANES)]
            rv = plsc.load_gather(rad_v, [zv])
            zrn_v[pl.ds(i, _LANES)] = plsc.bitcast(rv, jnp.int32)

        plsc.subcore_barrier()

        @pl.loop(0, n_slots, step=2)
        def _slots(t):
            for b in range(2):
                cid = (t + b) * _N_TILES + wid

                @pl.when(cid < n_chunks)
                def _():
                    @pl.when(t + b >= 2)
                    def _():
                        wait_out(b)

                    wait_in(b)

                    @plsc.parallel_loop(0, chunk, _LANES, unroll=8)
                    def _vec(i):
                        sv = e_bufs[b][0, pl.ds(i, _LANES)]
                        rv = e_bufs[b][1, pl.ds(i, _LANES)]
                        rs = plsc.bitcast(
                            plsc.load_gather(zrn_v, [sv]), jnp.float32
                        )
                        rr = plsc.bitcast(
                            plsc.load_gather(zrn_v, [rv]), jnp.float32
                        )
                        o_bufs[b][pl.ds(i, _LANES)] = rs + rr

                    start_out(cid, b)
                    nxt = cid + 2 * _N_TILES

                    @pl.when(nxt < n_chunks)
                    def _():
                        start_in(nxt, b)

        wait_out(0)
        wait_out(1)

    cp = pltpu.CompilerParams(needs_layout_passes=False)
    return pl.kernel(
        body,
        out_type=jax.ShapeDtypeStruct((n_edges,), jnp.float32),
        compiler_params=cp,
        mesh=plsc.VectorSubcoreMesh(
            core_axis_name="c", subcore_axis_name="s",
            num_cores=2, num_subcores=16,
        ),
        scratch_types=[
            pltpu.VMEM((n_nodes,), jnp.int32),
            pltpu.VMEM((rad_len,), jnp.float32),
            pltpu.VMEM((2, chunk), jnp.int32),
            pltpu.VMEM((2, chunk), jnp.int32),
            pltpu.VMEM((chunk,), jnp.float32),
            pltpu.VMEM((chunk,), jnp.float32),
            pltpu.SemaphoreType.DMA,
            pltpu.SemaphoreType.DMA,
            pltpu.SemaphoreType.DMA,
            pltpu.SemaphoreType.DMA,
            pltpu.SemaphoreType.DMA,
        ],
    )


def _tc_w_body(s_ref, d_ref, r0_ref, o_ref):
    t = d_ref[...] / r0_ref[...]
    lt = jnp.log(t)
    tq = jnp.exp(s_ref[1] * lt)
    tqp = jnp.exp(s_ref[2] * lt)
    atq = s_ref[0] * tq
    o_ref[...] = atq / (1.0 + tqp + atq)


@functools.lru_cache(maxsize=None)
def _build_tc_w(n_edges: int):
    cols = 128
    rows = n_edges // cols
    block_rows = 2000
    assert rows % block_rows == 0
    grid = rows // block_rows
    return pl.pallas_call(
        _tc_w_body,
        out_shape=jax.ShapeDtypeStruct((rows, cols), jnp.float32),
        grid=(grid,),
        in_specs=[
            pl.BlockSpec(memory_space=pltpu.SMEM),
            pl.BlockSpec((block_rows, cols), lambda i: (i, 0)),
            pl.BlockSpec((block_rows, cols), lambda i: (i, 0)),
        ],
        out_specs=pl.BlockSpec((block_rows, cols), lambda i: (i, 0)),
    )


def kernel(z, edge_distance, edge_index, q, p, covalent_radii):
    n_edges = edge_distance.shape[0]
    n_nodes = z.shape[0]
    pp = 2.0 * jax.nn.softplus(0.5 * p) + 1.0
    qq = 2.0 * jax.nn.softplus(0.5 * q) + 1.0
    a = -2.0 * (pp + qq - 2.0 * qq * pp) / (pp**2 + pp + qq**2 + qq)
    scalars = jnp.stack([a, qq, qq - pp]).astype(jnp.float32)

    rad_len = 128
    rad = jnp.concatenate(
        [covalent_radii.astype(jnp.float32),
         jnp.zeros((rad_len - covalent_radii.shape[0],), jnp.float32)]
    )
    r0 = _build_sc_r0(n_edges, n_nodes, rad_len)(edge_index, z, rad)

    cols = 128
    d2 = edge_distance.reshape(n_edges // cols, cols)
    r02 = r0.reshape(n_edges // cols, cols)
    w = _build_tc_w(n_edges)(scalars, d2, r02)
    return w.reshape(n_edges)
